# Initial kernel scaffold; baseline (speedup 1.0000x reference)
import jax
import jax.numpy as jnp
from jax.experimental import pallas as pl


def kernel(text_feat, text_mask, img_feat):
    raise NotImplementedError("write your pallas kernel here")


# speedup vs baseline: 34.9118x; 34.9118x over previous
import functools

import jax
import jax.numpy as jnp
from jax.experimental import pallas as pl

NQ = 10
CHUNK = 1024
NEG = float("-inf")


def _topk_rows(x, k):
    r = x.shape[0]
    iota = jax.lax.broadcasted_iota(jnp.int32, x.shape, 0)
    outs = []
    for _ in range(k):
        m = jnp.max(x, axis=0, keepdims=True)
        cand = jnp.where(x == m, iota, jnp.int32(r))
        amin = jnp.min(cand, axis=0, keepdims=True)
        x = jnp.where(iota == amin, NEG, x)
        outs.append(m)
    return jnp.concatenate(outs, axis=0)


def _body(x_ref, o_ref):
    n = pl.program_id(1)

    @pl.when(n == 0)
    def _init():
        o_ref[0] = jnp.full((NQ, o_ref.shape[2]), NEG, jnp.float32)

    prev = o_ref[0]
    chunk = x_ref[0]
    merged = jnp.concatenate([prev, chunk], axis=0)
    o_ref[0] = _topk_rows(merged, NQ)


def kernel(text_feat, text_mask, img_feat):
    b, n, c = text_feat.shape
    grid = (b, n // CHUNK)
    out = pl.pallas_call(
        _body,
        grid=grid,
        in_specs=[pl.BlockSpec((1, CHUNK, c), lambda i, j: (i, j, 0))],
        out_specs=pl.BlockSpec((1, NQ, c), lambda i, j: (i, 0, 0)),
        out_shape=jax.ShapeDtypeStruct((b, NQ, c), jnp.float32),
    )(text_feat)
    return out
